# Initial kernel scaffold; baseline (speedup 1.0000x reference)
import jax
import jax.numpy as jnp
from jax.experimental import pallas as pl


def kernel(bbox, x_table, y_table, ln_gamma, ln_beta, W, b):
    raise NotImplementedError("write your pallas kernel here")


# speedup vs baseline: 1.5461x; 1.5461x over previous
import functools

import jax
import jax.numpy as jnp
from jax import lax
from jax.experimental import pallas as pl
from jax.experimental.pallas import tpu as pltpu
from jax.experimental.pallas import tpu_sc as plsc

MAX_POS = 1024
HIDDEN = 768
EPS = 1e-12

NC = 2
NS = 16
NW = NC * NS

CHUNK = 128


def _sc_gather(tbl, idx_all, n_rows):
    rows_per_w = n_rows // NW
    n_chunks = rows_per_w // CHUNK
    mesh = plsc.VectorSubcoreMesh(core_axis_name="c", subcore_axis_name="s")

    @functools.partial(
        pl.kernel,
        out_type=jax.ShapeDtypeStruct((n_rows, HIDDEN), jnp.float32),
        mesh=mesh,
        scratch_types=[
            pltpu.VMEM((CHUNK,), jnp.int32),
            pltpu.VMEM((CHUNK, HIDDEN), jnp.float32),
            pltpu.SemaphoreType.DMA,
        ],
    )
    def k(tbl_hbm, idx_hbm, out_hbm, idx_v, rows_v, sem):
        wid = lax.axis_index("s") * NC + lax.axis_index("c")
        base = wid * rows_per_w
        for c in range(n_chunks):
            off = base + c * CHUNK
            pltpu.sync_copy(idx_hbm.at[pl.ds(off, CHUNK)], idx_v)
            pltpu.async_copy(tbl_hbm.at[idx_v], rows_v, sem).wait()
            pltpu.sync_copy(rows_v, out_hbm.at[pl.ds(off, CHUNK)])

    return k(tbl, idx_all)


BT = 512


def _tc_body(rows_ref, g_ref, bt_ref, wt_ref, b_ref, out_ref):
    r = rows_ref[...]
    emb = (r[0] + r[1]) + (r[2] + r[3])
    mean = jnp.mean(emb, axis=-1, keepdims=True)
    d = emb - mean
    var = jnp.mean(d * d, axis=-1, keepdims=True)
    nrm = d * lax.rsqrt(var + EPS) * g_ref[...] + bt_ref[...]
    out_ref[...] = (
        jnp.dot(nrm, wt_ref[...], preferred_element_type=jnp.float32) + b_ref[...]
    )


def _tc_ln_mlp(rows, gamma, beta, w_t, b):
    n_tok = rows.shape[1]
    return pl.pallas_call(
        _tc_body,
        grid=(n_tok // BT,),
        in_specs=[
            pl.BlockSpec((4, BT, HIDDEN), lambda i: (0, i, 0)),
            pl.BlockSpec((1, HIDDEN), lambda i: (0, 0)),
            pl.BlockSpec((1, HIDDEN), lambda i: (0, 0)),
            pl.BlockSpec((HIDDEN, HIDDEN), lambda i: (0, 0)),
            pl.BlockSpec((1, HIDDEN), lambda i: (0, 0)),
        ],
        out_specs=pl.BlockSpec((BT, HIDDEN), lambda i: (i, 0)),
        out_shape=jax.ShapeDtypeStruct((n_tok, HIDDEN), jnp.float32),
    )(rows, gamma, beta, w_t, b)


def kernel(bbox, x_table, y_table, ln_gamma, ln_beta, W, b):
    batch, seq, _ = bbox.shape
    n_tok = batch * seq
    idx = bbox.reshape(n_tok, 4).astype(jnp.int32)
    idx_all = jnp.concatenate(
        [idx[:, 0], idx[:, 1] + MAX_POS, idx[:, 2], idx[:, 3] + MAX_POS], axis=0
    )
    tbl = jnp.concatenate([x_table, y_table], axis=0)
    rows = _sc_gather(tbl, idx_all, 4 * n_tok)
    rows = rows.reshape(4, n_tok, HIDDEN)
    out = _tc_ln_mlp(
        rows,
        ln_gamma.reshape(1, HIDDEN),
        ln_beta.reshape(1, HIDDEN),
        W.T,
        b.reshape(1, HIDDEN),
    )
    return out.reshape(batch, seq, HIDDEN)
